# Initial kernel scaffold; baseline (speedup 1.0000x reference)
import jax
import jax.numpy as jnp
from jax.experimental import pallas as pl


def kernel(x, edge_index, batch, W_embed, b_embed, W_gcn, b_gcn, Wc1, bc1, Wc2, bc2):
    raise NotImplementedError("write your pallas kernel here")


# speedup vs baseline: 10.4786x; 10.4786x over previous
import functools

import jax
import jax.numpy as jnp
from jax import lax
from jax.experimental import pallas as pl
from jax.experimental.pallas import tpu as pltpu
from jax.experimental.pallas import tpu_sc as plsc

NN = 50000
EE = 800000
HH = 64
HALF = 32
GG = 64
NT = 16
NC = 2

BM = 2000
NBLK = NN // BM

DEG_PER_W = EE // (NC * NT)
DEG_FULL = DEG_PER_W // 128
DEG_TAIL = DEG_PER_W - DEG_FULL * 128

AGG_PER_T = EE // NT
AGG_FULL = AGG_PER_T // 128
AGG_TAIL = AGG_PER_T - AGG_FULL * 128

ROWS_PER_T = NN // NT

def _zero_vmem_2d(buf, rows):
    def body(i, _):
        buf[i, pl.ds(0, 16)] = jnp.zeros((16,), jnp.float32)
        buf[i, pl.ds(16, 16)] = jnp.zeros((16,), jnp.float32)
        return 0
    lax.fori_loop(0, rows, body, 0)


def _zero_vmem_1d(buf, n16):
    def body(i, _):
        buf[pl.ds(i * 16, 16)] = jnp.zeros((16,), jnp.float32)
        return 0
    lax.fori_loop(0, n16, body, 0)


def _sc_deg_body(ei, out0, out1, idx, idx_t, ones, zbuf, acc):
    c = lax.axis_index("c")
    s = lax.axis_index("s")
    _zero_vmem_1d(zbuf, 200)
    def ob(i, _):
        ones[pl.ds(i * 16, 16)] = jnp.ones((16,), jnp.float32)
        return 0
    lax.fori_loop(0, 8, ob, 0)
    @pl.when(s < NT - 1)
    def _():
        pltpu.sync_copy(zbuf, acc.at[pl.ds(s * 3200, 3200)])
    @pl.when(s == NT - 1)
    def _():
        pltpu.sync_copy(zbuf.at[pl.ds(0, 2000)], acc.at[pl.ds(48000, 2000)])
    plsc.subcore_barrier()

    wbase = (c * NT + s) * DEG_PER_W

    def chunk(k, _):
        pltpu.sync_copy(ei.at[pl.ds(EE + wbase + k * 128, 128)], idx)
        pltpu.sync_copy(ones, acc.at[idx], add=True)
        return 0
    lax.fori_loop(0, DEG_FULL, chunk, 0)
    pltpu.sync_copy(ei.at[pl.ds(EE + wbase + DEG_FULL * 128, DEG_TAIL)], idx_t)
    pltpu.sync_copy(ones.at[pl.ds(0, DEG_TAIL)], acc.at[idx_t], add=True)
    plsc.subcore_barrier()

    def wb(out):
        @pl.when(s < NT - 1)
        def _():
            pltpu.sync_copy(acc.at[pl.ds(s * 3200, 3200)], zbuf)
            pltpu.sync_copy(zbuf, out.at[pl.ds(s * 3200, 3200)])
        @pl.when(s == NT - 1)
        def _():
            pltpu.sync_copy(acc.at[pl.ds(48000, 2000)], zbuf.at[pl.ds(0, 2000)])
            pltpu.sync_copy(zbuf.at[pl.ds(0, 2000)], out.at[pl.ds(48000, 2000)])
    @pl.when(c == 0)
    def _():
        wb(out0)
    @pl.when(c == 1)
    def _():
        wb(out1)


def _sc_agg_body(ei, msA, msB, outA, outB,
                 idx_s, idx_d, idx_st, idx_dt, rows, rows_t, zbuf, wbuf, acc, sem):
    c = lax.axis_index("c")
    s = lax.axis_index("s")
    _zero_vmem_2d(zbuf, 125)
    r0 = s * ROWS_PER_T
    def zb(j, _):
        pltpu.sync_copy(zbuf, acc.at[pl.ds(r0 + j * 125, 125), :])
        return 0
    lax.fori_loop(0, ROWS_PER_T // 125, zb, 0)
    plsc.subcore_barrier()

    ebase = s * AGG_PER_T

    def chunk(k, _):
        b = ebase + k * 128
        pltpu.sync_copy(ei.at[pl.ds(b, 128)], idx_s)
        pltpu.sync_copy(ei.at[pl.ds(EE + b, 128)], idx_d)
        @pl.when(c == 0)
        def _():
            pltpu.async_copy(msA.at[idx_s], rows, sem).wait()
        @pl.when(c == 1)
        def _():
            pltpu.async_copy(msB.at[idx_s], rows, sem).wait()
        pltpu.sync_copy(rows, acc.at[idx_d], add=True)
        return 0
    lax.fori_loop(0, AGG_FULL, chunk, 0)

    bt = ebase + AGG_FULL * 128
    pltpu.sync_copy(ei.at[pl.ds(bt, AGG_TAIL)], idx_st)
    pltpu.sync_copy(ei.at[pl.ds(EE + bt, AGG_TAIL)], idx_dt)
    @pl.when(c == 0)
    def _():
        pltpu.async_copy(msA.at[idx_st], rows_t, sem).wait()
    @pl.when(c == 1)
    def _():
        pltpu.async_copy(msB.at[idx_st], rows_t, sem).wait()
    pltpu.sync_copy(rows_t, acc.at[idx_dt], add=True)
    plsc.subcore_barrier()

    def wb(out):
        npieces = jnp.where(s < NT - 1, 8, 5)
        def body(j, _):
            rr = s * 3200 + j * 400
            pltpu.sync_copy(acc.at[pl.ds(rr, 400), :], wbuf)
            pltpu.sync_copy(wbuf, out.at[pl.ds(rr, 400), :])
            return 0
        lax.fori_loop(0, npieces, body, 0)
    @pl.when(c == 0)
    def _():
        wb(outA)
    @pl.when(c == 1)
    def _():
        wb(outB)


@functools.lru_cache(maxsize=None)
def _build_sc():
    mesh = plsc.VectorSubcoreMesh(core_axis_name="c", subcore_axis_name="s")
    params = pltpu.CompilerParams(use_tc_tiling_on_sc=False)
    sc_deg = pl.kernel(
        _sc_deg_body,
        out_type=[jax.ShapeDtypeStruct((NN,), jnp.float32) for _ in range(NC)],
        mesh=mesh,
        compiler_params=params,
        scratch_types=[
            pltpu.VMEM((128,), jnp.int32),
            pltpu.VMEM((DEG_TAIL,), jnp.int32),
            pltpu.VMEM((128,), jnp.float32),
            pltpu.VMEM((3200,), jnp.float32),
            pltpu.VMEM_SHARED((NN,), jnp.float32),
        ],
    )
    sc_agg = pl.kernel(
        _sc_agg_body,
        out_type=[jax.ShapeDtypeStruct((NN, HALF), jnp.float32) for _ in range(NC)],
        mesh=mesh,
        compiler_params=params,
        scratch_types=[
            pltpu.VMEM((128,), jnp.int32),
            pltpu.VMEM((128,), jnp.int32),
            pltpu.VMEM((AGG_TAIL,), jnp.int32),
            pltpu.VMEM((AGG_TAIL,), jnp.int32),
            pltpu.VMEM((128, HALF), jnp.float32),
            pltpu.VMEM((AGG_TAIL, HALF), jnp.float32),
            pltpu.VMEM((125, HALF), jnp.float32),
            pltpu.VMEM((400, HALF), jnp.float32),
            pltpu.VMEM_SHARED((NN, HALF), jnp.float32),
            pltpu.SemaphoreType.DMA,
        ],
    )
    return sc_deg, sc_agg


def _stage_a_body(x_ref, d0_ref, d1_ref, we_ref, be_ref, w0_ref,
                  msA_ref, msB_ref, dinv_ref):
    deg = d0_ref[...] + d1_ref[...] + 1.0
    dinv = lax.rsqrt(deg)
    h0 = x_ref[...] * we_ref[...] + be_ref[...]
    m = jnp.dot(h0, w0_ref[...], preferred_element_type=jnp.float32)
    ms = dinv * m
    msA_ref[...] = ms[:, :HALF]
    msB_ref[...] = ms[:, HALF:]
    dinv_ref[...] = dinv


_col = pl.BlockSpec((BM, 1), lambda i: (i, 0))
_hhalf = pl.BlockSpec((BM, HALF), lambda i: (i, 0))
_full = lambda shape: pl.BlockSpec(shape, lambda i: tuple(0 for _ in shape))

_stage_a = pl.pallas_call(
    _stage_a_body,
    grid=(NBLK,),
    in_specs=[_col, _col, _col, _full((1, HH)), _full((1, HH)), _full((HH, HH))],
    out_specs=[_hhalf, _hhalf, _col],
    out_shape=[
        jax.ShapeDtypeStruct((NN, HALF), jnp.float32),
        jax.ShapeDtypeStruct((NN, HALF), jnp.float32),
        jax.ShapeDtypeStruct((NN, 1), jnp.float32),
    ],
)


def _stage_b_body(aA_ref, aB_ref, mA_ref, mB_ref, dinv_ref, b_ref, w_ref,
                  oA_ref, oB_ref):
    agg = jnp.concatenate([aA_ref[...], aB_ref[...]], axis=1)
    msp = jnp.concatenate([mA_ref[...], mB_ref[...]], axis=1)
    dinv = dinv_ref[...]
    h = jnp.maximum(dinv * (agg + msp) + b_ref[...], 0.0)
    ms = dinv * jnp.dot(h, w_ref[...], preferred_element_type=jnp.float32)
    oA_ref[...] = ms[:, :HALF]
    oB_ref[...] = ms[:, HALF:]


_stage_b = pl.pallas_call(
    _stage_b_body,
    grid=(NBLK,),
    in_specs=[_hhalf, _hhalf, _hhalf, _hhalf, _col, _full((1, HH)), _full((HH, HH))],
    out_specs=[_hhalf, _hhalf],
    out_shape=[
        jax.ShapeDtypeStruct((NN, HALF), jnp.float32),
        jax.ShapeDtypeStruct((NN, HALF), jnp.float32),
    ],
)


def _stage_c_body(aA_ref, aB_ref, mA_ref, mB_ref, dinv_ref, b_ref, batch_ref,
                  wc1_ref, bc1_ref, wc2_ref, bc2_ref, out_ref, gsum, cnt):
    i = pl.program_id(0)

    @pl.when(i == 0)
    def _():
        gsum[...] = jnp.zeros_like(gsum)
        cnt[...] = jnp.zeros_like(cnt)

    agg = jnp.concatenate([aA_ref[...], aB_ref[...]], axis=1)
    msp = jnp.concatenate([mA_ref[...], mB_ref[...]], axis=1)
    h = jnp.maximum(dinv_ref[...] * (agg + msp) + b_ref[...], 0.0)
    gid = batch_ref[...]
    oh = (gid == lax.broadcasted_iota(jnp.int32, (BM, GG), 1)).astype(jnp.float32)
    dn = (((0,), (0,)), ((), ()))
    gsum[...] += lax.dot_general(oh, h, dn, preferred_element_type=jnp.float32)
    cnt[...] += lax.dot_general(oh, jnp.ones((BM, 1), jnp.float32), dn,
                                preferred_element_type=jnp.float32)

    @pl.when(i == NBLK - 1)
    def _():
        gmean = gsum[...] / jnp.maximum(cnt[...], 1.0)
        z = jnp.maximum(jnp.dot(gmean, wc1_ref[...],
                                preferred_element_type=jnp.float32) + bc1_ref[...], 0.0)
        o = jnp.dot(z, wc2_ref[...], preferred_element_type=jnp.float32) + bc2_ref[...]
        out_ref[...] = jax.nn.sigmoid(o)


_stage_c = pl.pallas_call(
    _stage_c_body,
    grid=(NBLK,),
    in_specs=[_hhalf, _hhalf, _hhalf, _hhalf, _col, _full((1, HH)), _col,
              _full((HH, HALF)), _full((1, HALF)), _full((HALF, 1)), _full((1, 1))],
    out_specs=pl.BlockSpec((GG, 1), lambda i: (0, 0)),
    out_shape=jax.ShapeDtypeStruct((GG, 1), jnp.float32),
    scratch_shapes=[pltpu.VMEM((GG, HH), jnp.float32), pltpu.VMEM((GG, 1), jnp.float32)],
)


def kernel(x, edge_index, batch, W_embed, b_embed, W_gcn, b_gcn, Wc1, bc1, Wc2, bc2):
    _sc_deg, _sc_agg = _build_sc()
    ei_flat = edge_index.reshape(2 * EE)
    deg0, deg1 = _sc_deg(ei_flat)
    msA, msB, dinv = _stage_a(
        x, deg0.reshape(NN, 1), deg1.reshape(NN, 1),
        W_embed.reshape(1, HH), b_embed.reshape(1, HH), W_gcn[0])
    for i in (1, 2):
        aggA, aggB = _sc_agg(ei_flat, msA, msB)
        msA, msB = _stage_b(aggA, aggB, msA, msB, dinv,
                            b_gcn[i - 1].reshape(1, HH), W_gcn[i])
    aggA, aggB = _sc_agg(ei_flat, msA, msB)
    out = _stage_c(aggA, aggB, msA, msB, dinv, b_gcn[2].reshape(1, HH),
                   batch.reshape(NN, 1), Wc1, bc1.reshape(1, HALF),
                   Wc2, bc2.reshape(1, 1))
    return out.reshape(GG)


# speedup vs baseline: 25.3597x; 2.4201x over previous
import functools

import jax
import jax.numpy as jnp
from jax import lax
from jax.experimental import pallas as pl
from jax.experimental.pallas import tpu as pltpu
from jax.experimental.pallas import tpu_sc as plsc

NN = 50000
EE = 800000
HH = 64
HALF = 32
GG = 64
NT = 16
NC = 2

BM = 2000
NBLK = NN // BM

DEG_PER_W = EE // (NC * NT)
DEG_FULL = DEG_PER_W // 128
DEG_TAIL = DEG_PER_W - DEG_FULL * 128

AGG_PER_T = EE // NT
AGG_FULL = AGG_PER_T // 128
AGG_TAIL = AGG_PER_T - AGG_FULL * 128

ROWS_PER_T = NN // NT

def _zero_vmem_2d(buf, rows):
    def body(i, _):
        buf[i, pl.ds(0, 16)] = jnp.zeros((16,), jnp.float32)
        buf[i, pl.ds(16, 16)] = jnp.zeros((16,), jnp.float32)
        return 0
    lax.fori_loop(0, rows, body, 0)


def _zero_vmem_1d(buf, n16):
    def body(i, _):
        buf[pl.ds(i * 16, 16)] = jnp.zeros((16,), jnp.float32)
        return 0
    lax.fori_loop(0, n16, body, 0)


def _sc_deg_body(ei, out0, out1, idx, idx_t, ones, zbuf, acc):
    c = lax.axis_index("c")
    s = lax.axis_index("s")
    _zero_vmem_1d(zbuf, 200)
    def ob(i, _):
        ones[pl.ds(i * 16, 16)] = jnp.ones((16,), jnp.float32)
        return 0
    lax.fori_loop(0, 8, ob, 0)
    @pl.when(s < NT - 1)
    def _():
        pltpu.sync_copy(zbuf, acc.at[pl.ds(s * 3200, 3200)])
    @pl.when(s == NT - 1)
    def _():
        pltpu.sync_copy(zbuf.at[pl.ds(0, 2000)], acc.at[pl.ds(48000, 2000)])
    plsc.subcore_barrier()

    wbase = (c * NT + s) * DEG_PER_W

    def chunk(k, _):
        pltpu.sync_copy(ei.at[pl.ds(EE + wbase + k * 128, 128)], idx)
        pltpu.sync_copy(ones, acc.at[idx], add=True)
        return 0
    lax.fori_loop(0, DEG_FULL, chunk, 0)
    pltpu.sync_copy(ei.at[pl.ds(EE + wbase + DEG_FULL * 128, DEG_TAIL)], idx_t)
    pltpu.sync_copy(ones.at[pl.ds(0, DEG_TAIL)], acc.at[idx_t], add=True)
    plsc.subcore_barrier()

    def wb(out):
        @pl.when(s < NT - 1)
        def _():
            pltpu.sync_copy(acc.at[pl.ds(s * 3200, 3200)], zbuf)
            pltpu.sync_copy(zbuf, out.at[pl.ds(s * 3200, 3200)])
        @pl.when(s == NT - 1)
        def _():
            pltpu.sync_copy(acc.at[pl.ds(48000, 2000)], zbuf.at[pl.ds(0, 2000)])
            pltpu.sync_copy(zbuf.at[pl.ds(0, 2000)], out.at[pl.ds(48000, 2000)])
    @pl.when(c == 0)
    def _():
        wb(out0)
    @pl.when(c == 1)
    def _():
        wb(out1)


NBUF = 4
NCHUNK = EE // 128
CPT = NCHUNK // NT
XTRA = NCHUNK - CPT * NT
SLAB = 15


def _sc_agg_body(ei, msA, msB, outA, outB,
                 sidx, didx, dbuf, rows, wbuf, acc, sem_g, sem_e):
    c = lax.axis_index("c")
    s = lax.axis_index("s")
    _zero_vmem_2d(wbuf, 200)
    r0 = s * ROWS_PER_T
    def zb(j, _):
        pltpu.sync_copy(wbuf, acc.at[pl.ds(r0 + j * 200, 200), :])
        return 0
    lax.fori_loop(0, 15, zb, 0)
    pltpu.sync_copy(wbuf.at[pl.ds(0, 125), :], acc.at[pl.ds(r0 + 3000, 125), :])
    plsc.subcore_barrier()

    cs = s * CPT + jnp.minimum(s, XTRA)

    def issue_gather(i):
        r = lax.rem(i, NBUF)
        @pl.when(c == 0)
        def _():
            pltpu.async_copy(msA.at[sidx.at[pl.ds(i * 128, 128)]], rows.at[r], sem_g.at[r])
        @pl.when(c == 1)
        def _():
            pltpu.async_copy(msB.at[sidx.at[pl.ds(i * 128, 128)]], rows.at[r], sem_g.at[r])

    def wait_gather(i):
        r = lax.rem(i, NBUF)
        pltpu.make_async_copy(msA.at[sidx.at[pl.ds(i * 128, 128)]], rows.at[r],
                              sem_g.at[r]).wait()

    def load_dbuf(i):
        for k in range(8):
            dbuf[pl.ds(k * 16, 16)] = didx[pl.ds(i * 128 + k * 16, 16)]

    def slab(j, _):
        cb = cs + j * SLAB
        pltpu.sync_copy(ei.at[pl.ds(cb * 128, SLAB * 128)], sidx)
        pltpu.sync_copy(ei.at[pl.ds(EE + cb * 128, SLAB * 128)], didx)
        for p in range(NBUF):
            issue_gather(p)
        def body(i, _):
            wait_gather(i)
            load_dbuf(i)
            pltpu.sync_copy(rows.at[lax.rem(i, NBUF)], acc.at[dbuf], add=True)
            @pl.when(i + NBUF < SLAB)
            def _():
                issue_gather(i + NBUF)
            return 0
        lax.fori_loop(0, SLAB, body, 0)
        return 0
    lax.fori_loop(0, CPT // SLAB, slab, 0)

    @pl.when(s < XTRA)
    def _():
        ce = cs + CPT
        pltpu.sync_copy(ei.at[pl.ds(ce * 128, 128)], sidx.at[pl.ds(0, 128)])
        pltpu.sync_copy(ei.at[pl.ds(EE + ce * 128, 128)], didx.at[pl.ds(0, 128)])
        @pl.when(c == 0)
        def _():
            pltpu.async_copy(msA.at[sidx.at[pl.ds(0, 128)]], rows.at[0], sem_e).wait()
        @pl.when(c == 1)
        def _():
            pltpu.async_copy(msB.at[sidx.at[pl.ds(0, 128)]], rows.at[0], sem_e).wait()
        load_dbuf(0)
        pltpu.sync_copy(rows.at[0], acc.at[dbuf], add=True)
    plsc.subcore_barrier()

    def wb(out):
        npieces = jnp.where(s < NT - 1, 16, 10)
        def body(j, _):
            rr = s * 3200 + j * 200
            pltpu.sync_copy(acc.at[pl.ds(rr, 200), :], wbuf)
            pltpu.sync_copy(wbuf, out.at[pl.ds(rr, 200), :])
            return 0
        lax.fori_loop(0, npieces, body, 0)
    @pl.when(c == 0)
    def _():
        wb(outA)
    @pl.when(c == 1)
    def _():
        wb(outB)


@functools.lru_cache(maxsize=None)
def _build_sc():
    mesh = plsc.VectorSubcoreMesh(core_axis_name="c", subcore_axis_name="s")
    params = pltpu.CompilerParams(use_tc_tiling_on_sc=False)
    sc_deg = pl.kernel(
        _sc_deg_body,
        out_type=[jax.ShapeDtypeStruct((NN,), jnp.float32) for _ in range(NC)],
        mesh=mesh,
        compiler_params=params,
        scratch_types=[
            pltpu.VMEM((128,), jnp.int32),
            pltpu.VMEM((DEG_TAIL,), jnp.int32),
            pltpu.VMEM((128,), jnp.float32),
            pltpu.VMEM((3200,), jnp.float32),
            pltpu.VMEM_SHARED((NN,), jnp.float32),
        ],
    )
    sc_agg = pl.kernel(
        _sc_agg_body,
        out_type=[jax.ShapeDtypeStruct((NN, HALF), jnp.float32) for _ in range(NC)],
        mesh=mesh,
        compiler_params=params,
        scratch_types=[
            pltpu.VMEM((SLAB * 128,), jnp.int32),
            pltpu.VMEM((SLAB * 128,), jnp.int32),
            pltpu.VMEM((128,), jnp.int32),
            pltpu.VMEM((NBUF, 128, HALF), jnp.float32),
            pltpu.VMEM((200, HALF), jnp.float32),
            pltpu.VMEM_SHARED((NN, HALF), jnp.float32),
            pltpu.SemaphoreType.DMA((NBUF,)),
            pltpu.SemaphoreType.DMA,
        ],
    )
    return sc_deg, sc_agg


def _stage_a_body(x_ref, d0_ref, d1_ref, we_ref, be_ref, w0_ref,
                  msA_ref, msB_ref, dinv_ref):
    deg = d0_ref[...] + d1_ref[...] + 1.0
    dinv = lax.rsqrt(deg)
    h0 = x_ref[...] * we_ref[...] + be_ref[...]
    m = jnp.dot(h0, w0_ref[...], preferred_element_type=jnp.float32)
    ms = dinv * m
    msA_ref[...] = ms[:, :HALF]
    msB_ref[...] = ms[:, HALF:]
    dinv_ref[...] = dinv


_col = pl.BlockSpec((BM, 1), lambda i: (i, 0))
_hhalf = pl.BlockSpec((BM, HALF), lambda i: (i, 0))
_full = lambda shape: pl.BlockSpec(shape, lambda i: tuple(0 for _ in shape))

_stage_a = pl.pallas_call(
    _stage_a_body,
    grid=(NBLK,),
    in_specs=[_col, _col, _col, _full((1, HH)), _full((1, HH)), _full((HH, HH))],
    out_specs=[_hhalf, _hhalf, _col],
    out_shape=[
        jax.ShapeDtypeStruct((NN, HALF), jnp.float32),
        jax.ShapeDtypeStruct((NN, HALF), jnp.float32),
        jax.ShapeDtypeStruct((NN, 1), jnp.float32),
    ],
)


def _stage_b_body(aA_ref, aB_ref, mA_ref, mB_ref, dinv_ref, b_ref, w_ref,
                  oA_ref, oB_ref):
    agg = jnp.concatenate([aA_ref[...], aB_ref[...]], axis=1)
    msp = jnp.concatenate([mA_ref[...], mB_ref[...]], axis=1)
    dinv = dinv_ref[...]
    h = jnp.maximum(dinv * (agg + msp) + b_ref[...], 0.0)
    ms = dinv * jnp.dot(h, w_ref[...], preferred_element_type=jnp.float32)
    oA_ref[...] = ms[:, :HALF]
    oB_ref[...] = ms[:, HALF:]


_stage_b = pl.pallas_call(
    _stage_b_body,
    grid=(NBLK,),
    in_specs=[_hhalf, _hhalf, _hhalf, _hhalf, _col, _full((1, HH)), _full((HH, HH))],
    out_specs=[_hhalf, _hhalf],
    out_shape=[
        jax.ShapeDtypeStruct((NN, HALF), jnp.float32),
        jax.ShapeDtypeStruct((NN, HALF), jnp.float32),
    ],
)


def _stage_c_body(aA_ref, aB_ref, mA_ref, mB_ref, dinv_ref, b_ref, batch_ref,
                  wc1_ref, bc1_ref, wc2_ref, bc2_ref, out_ref, gsum, cnt):
    i = pl.program_id(0)

    @pl.when(i == 0)
    def _():
        gsum[...] = jnp.zeros_like(gsum)
        cnt[...] = jnp.zeros_like(cnt)

    agg = jnp.concatenate([aA_ref[...], aB_ref[...]], axis=1)
    msp = jnp.concatenate([mA_ref[...], mB_ref[...]], axis=1)
    h = jnp.maximum(dinv_ref[...] * (agg + msp) + b_ref[...], 0.0)
    gid = batch_ref[...]
    oh = (gid == lax.broadcasted_iota(jnp.int32, (BM, GG), 1)).astype(jnp.float32)
    dn = (((0,), (0,)), ((), ()))
    gsum[...] += lax.dot_general(oh, h, dn, preferred_element_type=jnp.float32)
    cnt[...] += lax.dot_general(oh, jnp.ones((BM, 1), jnp.float32), dn,
                                preferred_element_type=jnp.float32)

    @pl.when(i == NBLK - 1)
    def _():
        gmean = gsum[...] / jnp.maximum(cnt[...], 1.0)
        z = jnp.maximum(jnp.dot(gmean, wc1_ref[...],
                                preferred_element_type=jnp.float32) + bc1_ref[...], 0.0)
        o = jnp.dot(z, wc2_ref[...], preferred_element_type=jnp.float32) + bc2_ref[...]
        out_ref[...] = jax.nn.sigmoid(o)


_stage_c = pl.pallas_call(
    _stage_c_body,
    grid=(NBLK,),
    in_specs=[_hhalf, _hhalf, _hhalf, _hhalf, _col, _full((1, HH)), _col,
              _full((HH, HALF)), _full((1, HALF)), _full((HALF, 1)), _full((1, 1))],
    out_specs=pl.BlockSpec((GG, 1), lambda i: (0, 0)),
    out_shape=jax.ShapeDtypeStruct((GG, 1), jnp.float32),
    scratch_shapes=[pltpu.VMEM((GG, HH), jnp.float32), pltpu.VMEM((GG, 1), jnp.float32)],
)


def kernel(x, edge_index, batch, W_embed, b_embed, W_gcn, b_gcn, Wc1, bc1, Wc2, bc2):
    _sc_deg, _sc_agg = _build_sc()
    ei_flat = edge_index.reshape(2 * EE)
    deg0, deg1 = _sc_deg(ei_flat)
    msA, msB, dinv = _stage_a(
        x, deg0.reshape(NN, 1), deg1.reshape(NN, 1),
        W_embed.reshape(1, HH), b_embed.reshape(1, HH), W_gcn[0])
    for i in (1, 2):
        aggA, aggB = _sc_agg(ei_flat, msA, msB)
        msA, msB = _stage_b(aggA, aggB, msA, msB, dinv,
                            b_gcn[i - 1].reshape(1, HH), W_gcn[i])
    aggA, aggB = _sc_agg(ei_flat, msA, msB)
    out = _stage_c(aggA, aggB, msA, msB, dinv, b_gcn[2].reshape(1, HH),
                   batch.reshape(NN, 1), Wc1, bc1.reshape(1, HALF),
                   Wc2, bc2.reshape(1, 1))
    return out.reshape(GG)


# speedup vs baseline: 27.3005x; 1.0765x over previous
import functools

import jax
import jax.numpy as jnp
from jax import lax
from jax.experimental import pallas as pl
from jax.experimental.pallas import tpu as pltpu
from jax.experimental.pallas import tpu_sc as plsc

NN = 50000
EE = 800000
HH = 64
HALF = 32
GG = 64
NT = 16
NC = 2

BM = 2000
NBLK = NN // BM

DEG_PER_W = EE // (NC * NT)
DEG_FULL = DEG_PER_W // 128
DEG_TAIL = DEG_PER_W - DEG_FULL * 128

AGG_PER_T = EE // NT
AGG_FULL = AGG_PER_T // 128
AGG_TAIL = AGG_PER_T - AGG_FULL * 128

ROWS_PER_T = NN // NT

def _zero_vmem_2d(buf, rows):
    def body(i, _):
        buf[i, pl.ds(0, 16)] = jnp.zeros((16,), jnp.float32)
        buf[i, pl.ds(16, 16)] = jnp.zeros((16,), jnp.float32)
        return 0
    lax.fori_loop(0, rows, body, 0)


def _zero_vmem_1d(buf, n16):
    def body(i, _):
        buf[pl.ds(i * 16, 16)] = jnp.zeros((16,), jnp.float32)
        return 0
    lax.fori_loop(0, n16, body, 0)


def _sc_deg_body(ei, out0, out1, idx, idx_t, ones, zbuf, acc):
    c = lax.axis_index("c")
    s = lax.axis_index("s")
    _zero_vmem_1d(zbuf, 200)
    def ob(i, _):
        ones[pl.ds(i * 16, 16)] = jnp.ones((16,), jnp.float32)
        return 0
    lax.fori_loop(0, 8, ob, 0)
    @pl.when(s < NT - 1)
    def _():
        pltpu.sync_copy(zbuf, acc.at[pl.ds(s * 3200, 3200)])
    @pl.when(s == NT - 1)
    def _():
        pltpu.sync_copy(zbuf.at[pl.ds(0, 2000)], acc.at[pl.ds(48000, 2000)])
    plsc.subcore_barrier()

    wbase = (c * NT + s) * DEG_PER_W

    def dslab(j, _):
        pltpu.sync_copy(ei.at[pl.ds(EE + wbase + j * 1920, 1920)], idx)
        def chunk(k, _):
            pltpu.sync_copy(ones, acc.at[idx.at[pl.ds(k * 128, 128)]], add=True)
            return 0
        lax.fori_loop(0, 15, chunk, 0)
        return 0
    lax.fori_loop(0, 13, dslab, 0)
    pltpu.sync_copy(ei.at[pl.ds(EE + wbase + 24960, DEG_TAIL)], idx_t)
    pltpu.sync_copy(ones.at[pl.ds(0, DEG_TAIL)], acc.at[idx_t], add=True)
    plsc.subcore_barrier()

    def wb(out):
        @pl.when(s < NT - 1)
        def _():
            pltpu.sync_copy(acc.at[pl.ds(s * 3200, 3200)], zbuf)
            pltpu.sync_copy(zbuf, out.at[pl.ds(s * 3200, 3200)])
        @pl.when(s == NT - 1)
        def _():
            pltpu.sync_copy(acc.at[pl.ds(48000, 2000)], zbuf.at[pl.ds(0, 2000)])
            pltpu.sync_copy(zbuf.at[pl.ds(0, 2000)], out.at[pl.ds(48000, 2000)])
    @pl.when(c == 0)
    def _():
        wb(out0)
    @pl.when(c == 1)
    def _():
        wb(out1)


NBUF = 4
NCHUNK = EE // 128
CPT = NCHUNK // NT
XTRA = NCHUNK - CPT * NT
SLAB = 15


def _sc_agg_body(ei, msA, msB, outA, outB,
                 sidx, didx, dbuf, rows, wbuf, acc, sem_g, sem_s, sem_e):
    c = lax.axis_index("c")
    s = lax.axis_index("s")
    _zero_vmem_2d(wbuf, 200)
    r0 = s * ROWS_PER_T
    def zb(j, _):
        pltpu.sync_copy(wbuf, acc.at[pl.ds(r0 + j * 200, 200), :])
        return 0
    lax.fori_loop(0, 15, zb, 0)
    pltpu.sync_copy(wbuf.at[pl.ds(0, 125), :], acc.at[pl.ds(r0 + 3000, 125), :])
    plsc.subcore_barrier()

    cs = s * CPT + jnp.minimum(s, XTRA)

    def issue_gather(i):
        r = lax.rem(i, NBUF)
        @pl.when(c == 0)
        def _():
            pltpu.async_copy(msA.at[sidx.at[pl.ds(i * 128, 128)]], rows.at[r], sem_g.at[r])
        @pl.when(c == 1)
        def _():
            pltpu.async_copy(msB.at[sidx.at[pl.ds(i * 128, 128)]], rows.at[r], sem_g.at[r])

    def wait_gather(i):
        r = lax.rem(i, NBUF)
        pltpu.make_async_copy(msA.at[sidx.at[pl.ds(i * 128, 128)]], rows.at[r],
                              sem_g.at[r]).wait()

    def load_dbuf(i):
        r = lax.rem(i, NBUF)
        for k in range(8):
            dbuf[r, pl.ds(k * 16, 16)] = didx[pl.ds(i * 128 + k * 16, 16)]

    def issue_scatter(i):
        r = lax.rem(i, NBUF)
        pltpu.async_copy(rows.at[r], acc.at[dbuf.at[r]], sem_s.at[r], add=True)

    def wait_scatter(i):
        r = lax.rem(i, NBUF)
        pltpu.make_async_copy(rows.at[r], acc.at[dbuf.at[r]], sem_s.at[r]).wait()

    def slab(j, _):
        cb = cs + j * SLAB
        pltpu.sync_copy(ei.at[pl.ds(cb * 128, SLAB * 128)], sidx)
        pltpu.sync_copy(ei.at[pl.ds(EE + cb * 128, SLAB * 128)], didx)
        for p in range(NBUF):
            issue_gather(p)
        def body(i, _):
            wait_gather(i)
            load_dbuf(i)
            issue_scatter(i)
            @pl.when(i + NBUF < SLAB)
            def _():
                wait_scatter(i)
                issue_gather(i + NBUF)
            return 0
        lax.fori_loop(0, SLAB, body, 0)
        for p in range(SLAB - NBUF, SLAB):
            wait_scatter(p)
        return 0
    lax.fori_loop(0, CPT // SLAB, slab, 0)

    @pl.when(s < XTRA)
    def _():
        ce = cs + CPT
        pltpu.sync_copy(ei.at[pl.ds(ce * 128, 128)], sidx.at[pl.ds(0, 128)])
        pltpu.sync_copy(ei.at[pl.ds(EE + ce * 128, 128)], didx.at[pl.ds(0, 128)])
        @pl.when(c == 0)
        def _():
            pltpu.async_copy(msA.at[sidx.at[pl.ds(0, 128)]], rows.at[0], sem_e).wait()
        @pl.when(c == 1)
        def _():
            pltpu.async_copy(msB.at[sidx.at[pl.ds(0, 128)]], rows.at[0], sem_e).wait()
        load_dbuf(0)
        pltpu.sync_copy(rows.at[0], acc.at[dbuf.at[0]], add=True)
    plsc.subcore_barrier()

    def wb(out):
        npieces = jnp.where(s < NT - 1, 16, 10)
        def body(j, _):
            rr = s * 3200 + j * 200
            pltpu.sync_copy(acc.at[pl.ds(rr, 200), :], wbuf)
            pltpu.sync_copy(wbuf, out.at[pl.ds(rr, 200), :])
            return 0
        lax.fori_loop(0, npieces, body, 0)
    @pl.when(c == 0)
    def _():
        wb(outA)
    @pl.when(c == 1)
    def _():
        wb(outB)


@functools.lru_cache(maxsize=None)
def _build_sc():
    mesh = plsc.VectorSubcoreMesh(core_axis_name="c", subcore_axis_name="s")
    params = pltpu.CompilerParams(use_tc_tiling_on_sc=False)
    sc_deg = pl.kernel(
        _sc_deg_body,
        out_type=[jax.ShapeDtypeStruct((NN,), jnp.float32) for _ in range(NC)],
        mesh=mesh,
        compiler_params=params,
        scratch_types=[
            pltpu.VMEM((1920,), jnp.int32),
            pltpu.VMEM((DEG_TAIL,), jnp.int32),
            pltpu.VMEM((128,), jnp.float32),
            pltpu.VMEM((3200,), jnp.float32),
            pltpu.VMEM_SHARED((NN,), jnp.float32),
        ],
    )
    sc_agg = pl.kernel(
        _sc_agg_body,
        out_type=[jax.ShapeDtypeStruct((NN, HALF), jnp.float32) for _ in range(NC)],
        mesh=mesh,
        compiler_params=params,
        scratch_types=[
            pltpu.VMEM((SLAB * 128,), jnp.int32),
            pltpu.VMEM((SLAB * 128,), jnp.int32),
            pltpu.VMEM((NBUF, 128), jnp.int32),
            pltpu.VMEM((NBUF, 128, HALF), jnp.float32),
            pltpu.VMEM((200, HALF), jnp.float32),
            pltpu.VMEM_SHARED((NN, HALF), jnp.float32),
            pltpu.SemaphoreType.DMA((NBUF,)),
            pltpu.SemaphoreType.DMA((NBUF,)),
            pltpu.SemaphoreType.DMA,
        ],
    )
    return sc_deg, sc_agg


def _stage_a_body(x_ref, d0_ref, d1_ref, we_ref, be_ref, w0_ref,
                  msA_ref, msB_ref, dinv_ref):
    deg = d0_ref[...] + d1_ref[...] + 1.0
    dinv = lax.rsqrt(deg)
    h0 = x_ref[...] * we_ref[...] + be_ref[...]
    m = jnp.dot(h0, w0_ref[...], preferred_element_type=jnp.float32)
    ms = dinv * m
    msA_ref[...] = ms[:, :HALF]
    msB_ref[...] = ms[:, HALF:]
    dinv_ref[...] = dinv


_col = pl.BlockSpec((BM, 1), lambda i: (i, 0))
_hhalf = pl.BlockSpec((BM, HALF), lambda i: (i, 0))
_full = lambda shape: pl.BlockSpec(shape, lambda i: tuple(0 for _ in shape))

_stage_a = pl.pallas_call(
    _stage_a_body,
    grid=(NBLK,),
    in_specs=[_col, _col, _col, _full((1, HH)), _full((1, HH)), _full((HH, HH))],
    out_specs=[_hhalf, _hhalf, _col],
    out_shape=[
        jax.ShapeDtypeStruct((NN, HALF), jnp.float32),
        jax.ShapeDtypeStruct((NN, HALF), jnp.float32),
        jax.ShapeDtypeStruct((NN, 1), jnp.float32),
    ],
)


def _stage_b_body(aA_ref, aB_ref, mA_ref, mB_ref, dinv_ref, b_ref, w_ref,
                  oA_ref, oB_ref):
    agg = jnp.concatenate([aA_ref[...], aB_ref[...]], axis=1)
    msp = jnp.concatenate([mA_ref[...], mB_ref[...]], axis=1)
    dinv = dinv_ref[...]
    h = jnp.maximum(dinv * (agg + msp) + b_ref[...], 0.0)
    ms = dinv * jnp.dot(h, w_ref[...], preferred_element_type=jnp.float32)
    oA_ref[...] = ms[:, :HALF]
    oB_ref[...] = ms[:, HALF:]


_stage_b = pl.pallas_call(
    _stage_b_body,
    grid=(NBLK,),
    in_specs=[_hhalf, _hhalf, _hhalf, _hhalf, _col, _full((1, HH)), _full((HH, HH))],
    out_specs=[_hhalf, _hhalf],
    out_shape=[
        jax.ShapeDtypeStruct((NN, HALF), jnp.float32),
        jax.ShapeDtypeStruct((NN, HALF), jnp.float32),
    ],
)


def _stage_c_body(aA_ref, aB_ref, mA_ref, mB_ref, dinv_ref, b_ref, batch_ref,
                  wc1_ref, bc1_ref, wc2_ref, bc2_ref, out_ref, gsum, cnt):
    i = pl.program_id(0)

    @pl.when(i == 0)
    def _():
        gsum[...] = jnp.zeros_like(gsum)
        cnt[...] = jnp.zeros_like(cnt)

    agg = jnp.concatenate([aA_ref[...], aB_ref[...]], axis=1)
    msp = jnp.concatenate([mA_ref[...], mB_ref[...]], axis=1)
    h = jnp.maximum(dinv_ref[...] * (agg + msp) + b_ref[...], 0.0)
    gid = batch_ref[...]
    oh = (gid == lax.broadcasted_iota(jnp.int32, (BM, GG), 1)).astype(jnp.float32)
    dn = (((0,), (0,)), ((), ()))
    gsum[...] += lax.dot_general(oh, h, dn, preferred_element_type=jnp.float32)
    cnt[...] += lax.dot_general(oh, jnp.ones((BM, 1), jnp.float32), dn,
                                preferred_element_type=jnp.float32)

    @pl.when(i == NBLK - 1)
    def _():
        gmean = gsum[...] / jnp.maximum(cnt[...], 1.0)
        z = jnp.maximum(jnp.dot(gmean, wc1_ref[...],
                                preferred_element_type=jnp.float32) + bc1_ref[...], 0.0)
        o = jnp.dot(z, wc2_ref[...], preferred_element_type=jnp.float32) + bc2_ref[...]
        out_ref[...] = jax.nn.sigmoid(o)


_stage_c = pl.pallas_call(
    _stage_c_body,
    grid=(NBLK,),
    in_specs=[_hhalf, _hhalf, _hhalf, _hhalf, _col, _full((1, HH)), _col,
              _full((HH, HALF)), _full((1, HALF)), _full((HALF, 1)), _full((1, 1))],
    out_specs=pl.BlockSpec((GG, 1), lambda i: (0, 0)),
    out_shape=jax.ShapeDtypeStruct((GG, 1), jnp.float32),
    scratch_shapes=[pltpu.VMEM((GG, HH), jnp.float32), pltpu.VMEM((GG, 1), jnp.float32)],
)


def kernel(x, edge_index, batch, W_embed, b_embed, W_gcn, b_gcn, Wc1, bc1, Wc2, bc2):
    _sc_deg, _sc_agg = _build_sc()
    ei_flat = edge_index.reshape(2 * EE)
    deg0, deg1 = _sc_deg(ei_flat)
    msA, msB, dinv = _stage_a(
        x, deg0.reshape(NN, 1), deg1.reshape(NN, 1),
        W_embed.reshape(1, HH), b_embed.reshape(1, HH), W_gcn[0])
    for i in (1, 2):
        aggA, aggB = _sc_agg(ei_flat, msA, msB)
        msA, msB = _stage_b(aggA, aggB, msA, msB, dinv,
                            b_gcn[i - 1].reshape(1, HH), W_gcn[i])
    aggA, aggB = _sc_agg(ei_flat, msA, msB)
    out = _stage_c(aggA, aggB, msA, msB, dinv, b_gcn[2].reshape(1, HH),
                   batch.reshape(NN, 1), Wc1, bc1.reshape(1, HALF),
                   Wc2, bc2.reshape(1, 1))
    return out.reshape(GG)


# speedup vs baseline: 29.6309x; 1.0854x over previous
import functools

import jax
import jax.numpy as jnp
from jax import lax
from jax.experimental import pallas as pl
from jax.experimental.pallas import tpu as pltpu
from jax.experimental.pallas import tpu_sc as plsc

NN = 50000
EE = 800000
HH = 64
HALF = 32
GG = 64
NT = 16
NC = 2

BM = 2000
NBLK = NN // BM

DEG_PER_W = EE // (NC * NT)
DEG_FULL = DEG_PER_W // 128
DEG_TAIL = DEG_PER_W - DEG_FULL * 128

AGG_PER_T = EE // NT
AGG_FULL = AGG_PER_T // 128
AGG_TAIL = AGG_PER_T - AGG_FULL * 128

ROWS_PER_T = NN // NT

def _zero_vmem_2d(buf, rows):
    def body(i, _):
        buf[i, pl.ds(0, 16)] = jnp.zeros((16,), jnp.float32)
        buf[i, pl.ds(16, 16)] = jnp.zeros((16,), jnp.float32)
        return 0
    lax.fori_loop(0, rows, body, 0)


def _zero_vmem_1d(buf, n16):
    def body(i, _):
        buf[pl.ds(i * 16, 16)] = jnp.zeros((16,), jnp.float32)
        return 0
    lax.fori_loop(0, n16, body, 0)


def _sc_deg_body(ei, out0, out1, idx, idx_t, ones, zbuf, acc):
    c = lax.axis_index("c")
    s = lax.axis_index("s")
    _zero_vmem_1d(zbuf, 200)
    def ob(i, _):
        ones[pl.ds(i * 16, 16)] = jnp.ones((16,), jnp.float32)
        return 0
    lax.fori_loop(0, 8, ob, 0)
    @pl.when(s < NT - 1)
    def _():
        pltpu.sync_copy(zbuf, acc.at[pl.ds(s * 3200, 3200)])
    @pl.when(s == NT - 1)
    def _():
        pltpu.sync_copy(zbuf.at[pl.ds(0, 2000)], acc.at[pl.ds(48000, 2000)])
    plsc.subcore_barrier()

    wbase = (c * NT + s) * DEG_PER_W

    def dslab(j, _):
        pltpu.sync_copy(ei.at[pl.ds(EE + wbase + j * 1920, 1920)], idx)
        def chunk(k, _):
            pltpu.sync_copy(ones, acc.at[idx.at[pl.ds(k * 128, 128)]], add=True)
            return 0
        lax.fori_loop(0, 15, chunk, 0)
        return 0
    lax.fori_loop(0, 13, dslab, 0)
    pltpu.sync_copy(ei.at[pl.ds(EE + wbase + 24960, DEG_TAIL)], idx_t)
    pltpu.sync_copy(ones.at[pl.ds(0, DEG_TAIL)], acc.at[idx_t], add=True)
    plsc.subcore_barrier()

    def wb(out):
        @pl.when(s < NT - 1)
        def _():
            pltpu.sync_copy(acc.at[pl.ds(s * 3200, 3200)], zbuf)
            pltpu.sync_copy(zbuf, out.at[pl.ds(s * 3200, 3200)])
        @pl.when(s == NT - 1)
        def _():
            pltpu.sync_copy(acc.at[pl.ds(48000, 2000)], zbuf.at[pl.ds(0, 2000)])
            pltpu.sync_copy(zbuf.at[pl.ds(0, 2000)], out.at[pl.ds(48000, 2000)])
    @pl.when(c == 0)
    def _():
        wb(out0)
    @pl.when(c == 1)
    def _():
        wb(out1)


NBUF = 4
NCHUNK = EE // 128
CPT = NCHUNK // NT
XTRA = NCHUNK - CPT * NT
SLAB = 15


def _sc_agg_body(ei, msA, msB, outA, outB,
                 sidx, didx, dbuf, rows, zbuf, acc, sem_g, sem_s, sem_i, sem_e):
    c = lax.axis_index("c")
    s = lax.axis_index("s")
    _zero_vmem_2d(zbuf, 125)
    r0 = s * ROWS_PER_T
    def zb(j, _):
        pltpu.sync_copy(zbuf, acc.at[pl.ds(r0 + j * 125, 125), :])
        return 0
    lax.fori_loop(0, ROWS_PER_T // 125, zb, 0)
    plsc.subcore_barrier()

    cs = s * CPT + jnp.minimum(s, XTRA)
    NSLAB = CPT // SLAB

    def issue_slab(j):
        p = lax.rem(j, 2)
        cb = cs + j * SLAB
        pltpu.async_copy(ei.at[pl.ds(cb * 128, SLAB * 128)], sidx.at[p], sem_i.at[p])
        pltpu.async_copy(ei.at[pl.ds(EE + cb * 128, SLAB * 128)], didx.at[p], sem_i.at[p])

    def wait_slab(j):
        p = lax.rem(j, 2)
        cb = cs + j * SLAB
        pltpu.make_async_copy(ei.at[pl.ds(cb * 128, SLAB * 128)], sidx.at[p],
                              sem_i.at[p]).wait()
        pltpu.make_async_copy(ei.at[pl.ds(EE + cb * 128, SLAB * 128)], didx.at[p],
                              sem_i.at[p]).wait()

    def issue_gather(p, i):
        r = lax.rem(i, NBUF)
        @pl.when(c == 0)
        def _():
            pltpu.async_copy(msA.at[sidx.at[p, pl.ds(i * 128, 128)]], rows.at[r],
                             sem_g.at[r])
        @pl.when(c == 1)
        def _():
            pltpu.async_copy(msB.at[sidx.at[p, pl.ds(i * 128, 128)]], rows.at[r],
                             sem_g.at[r])

    def wait_gather(p, i):
        r = lax.rem(i, NBUF)
        pltpu.make_async_copy(msA.at[sidx.at[p, pl.ds(i * 128, 128)]], rows.at[r],
                              sem_g.at[r]).wait()

    def load_dbuf(p, i):
        r = lax.rem(i, NBUF)
        for k in range(8):
            dbuf[r, pl.ds(k * 16, 16)] = didx[p, pl.ds(i * 128 + k * 16, 16)]

    def issue_scatter(i):
        r = lax.rem(i, NBUF)
        pltpu.async_copy(rows.at[r], acc.at[dbuf.at[r]], sem_s.at[r], add=True)

    def wait_scatter(i):
        r = lax.rem(i, NBUF)
        pltpu.make_async_copy(rows.at[r], acc.at[dbuf.at[r]], sem_s.at[r]).wait()

    issue_slab(0)

    def slab(j, _):
        p = lax.rem(j, 2)
        wait_slab(j)
        @pl.when(j + 1 < NSLAB)
        def _():
            issue_slab(j + 1)
        for q in range(NBUF):
            issue_gather(p, q)
        def body(i, _):
            wait_gather(p, i)
            load_dbuf(p, i)
            issue_scatter(i)
            @pl.when(i + NBUF < SLAB)
            def _():
                wait_scatter(i)
                issue_gather(p, i + NBUF)
            return 0
        lax.fori_loop(0, SLAB, body, 0)
        for q in range(SLAB - NBUF, SLAB):
            wait_scatter(q)
        return 0
    lax.fori_loop(0, NSLAB, slab, 0)

    @pl.when(s < XTRA)
    def _():
        ce = cs + CPT
        pltpu.sync_copy(ei.at[pl.ds(ce * 128, 128)], sidx.at[0, pl.ds(0, 128)])
        pltpu.sync_copy(ei.at[pl.ds(EE + ce * 128, 128)], didx.at[0, pl.ds(0, 128)])
        @pl.when(c == 0)
        def _():
            pltpu.async_copy(msA.at[sidx.at[0, pl.ds(0, 128)]], rows.at[0], sem_e).wait()
        @pl.when(c == 1)
        def _():
            pltpu.async_copy(msB.at[sidx.at[0, pl.ds(0, 128)]], rows.at[0], sem_e).wait()
        load_dbuf(0, 0)
        pltpu.sync_copy(rows.at[0], acc.at[dbuf.at[0]], add=True)
    plsc.subcore_barrier()

    def wb(out):
        @pl.when(s < NT - 1)
        def _():
            pltpu.sync_copy(acc.at[pl.ds(s * 3200, 3200), :], out.at[pl.ds(s * 3200, 3200), :])
        @pl.when(s == NT - 1)
        def _():
            pltpu.sync_copy(acc.at[pl.ds(48000, 2000), :], out.at[pl.ds(48000, 2000), :])
    @pl.when(c == 0)
    def _():
        wb(outA)
    @pl.when(c == 1)
    def _():
        wb(outB)


@functools.lru_cache(maxsize=None)
def _build_sc():
    mesh = plsc.VectorSubcoreMesh(core_axis_name="c", subcore_axis_name="s")
    params = pltpu.CompilerParams(use_tc_tiling_on_sc=False)
    sc_deg = pl.kernel(
        _sc_deg_body,
        out_type=[jax.ShapeDtypeStruct((NN,), jnp.float32) for _ in range(NC)],
        mesh=mesh,
        compiler_params=params,
        scratch_types=[
            pltpu.VMEM((1920,), jnp.int32),
            pltpu.VMEM((DEG_TAIL,), jnp.int32),
            pltpu.VMEM((128,), jnp.float32),
            pltpu.VMEM((3200,), jnp.float32),
            pltpu.VMEM_SHARED((NN,), jnp.float32),
        ],
    )
    sc_agg = pl.kernel(
        _sc_agg_body,
        out_type=[jax.ShapeDtypeStruct((NN, HALF), jnp.float32) for _ in range(NC)],
        mesh=mesh,
        compiler_params=params,
        scratch_types=[
            pltpu.VMEM((2, SLAB * 128), jnp.int32),
            pltpu.VMEM((2, SLAB * 128), jnp.int32),
            pltpu.VMEM((NBUF, 128), jnp.int32),
            pltpu.VMEM((NBUF, 128, HALF), jnp.float32),
            pltpu.VMEM((125, HALF), jnp.float32),
            pltpu.VMEM_SHARED((NN, HALF), jnp.float32),
            pltpu.SemaphoreType.DMA((NBUF,)),
            pltpu.SemaphoreType.DMA((NBUF,)),
            pltpu.SemaphoreType.DMA((2,)),
            pltpu.SemaphoreType.DMA,
        ],
    )
    return sc_deg, sc_agg


def _stage_a_body(x_ref, d0_ref, d1_ref, we_ref, be_ref, w0_ref,
                  msA_ref, msB_ref, dinv_ref):
    deg = d0_ref[...] + d1_ref[...] + 1.0
    dinv = lax.rsqrt(deg)
    h0 = x_ref[...] * we_ref[...] + be_ref[...]
    m = jnp.dot(h0, w0_ref[...], preferred_element_type=jnp.float32)
    ms = dinv * m
    msA_ref[...] = ms[:, :HALF]
    msB_ref[...] = ms[:, HALF:]
    dinv_ref[...] = dinv


_col = pl.BlockSpec((BM, 1), lambda i: (i, 0))
_hhalf = pl.BlockSpec((BM, HALF), lambda i: (i, 0))
_full = lambda shape: pl.BlockSpec(shape, lambda i: tuple(0 for _ in shape))

_stage_a = pl.pallas_call(
    _stage_a_body,
    grid=(NBLK,),
    in_specs=[_col, _col, _col, _full((1, HH)), _full((1, HH)), _full((HH, HH))],
    out_specs=[_hhalf, _hhalf, _col],
    out_shape=[
        jax.ShapeDtypeStruct((NN, HALF), jnp.float32),
        jax.ShapeDtypeStruct((NN, HALF), jnp.float32),
        jax.ShapeDtypeStruct((NN, 1), jnp.float32),
    ],
)


def _stage_b_body(aA_ref, aB_ref, mA_ref, mB_ref, dinv_ref, b_ref, w_ref,
                  oA_ref, oB_ref):
    agg = jnp.concatenate([aA_ref[...], aB_ref[...]], axis=1)
    msp = jnp.concatenate([mA_ref[...], mB_ref[...]], axis=1)
    dinv = dinv_ref[...]
    h = jnp.maximum(dinv * (agg + msp) + b_ref[...], 0.0)
    ms = dinv * jnp.dot(h, w_ref[...], preferred_element_type=jnp.float32)
    oA_ref[...] = ms[:, :HALF]
    oB_ref[...] = ms[:, HALF:]


_stage_b = pl.pallas_call(
    _stage_b_body,
    grid=(NBLK,),
    in_specs=[_hhalf, _hhalf, _hhalf, _hhalf, _col, _full((1, HH)), _full((HH, HH))],
    out_specs=[_hhalf, _hhalf],
    out_shape=[
        jax.ShapeDtypeStruct((NN, HALF), jnp.float32),
        jax.ShapeDtypeStruct((NN, HALF), jnp.float32),
    ],
)


def _stage_c_body(aA_ref, aB_ref, mA_ref, mB_ref, dinv_ref, b_ref, batch_ref,
                  wc1_ref, bc1_ref, wc2_ref, bc2_ref, out_ref, gsum, cnt):
    i = pl.program_id(0)

    @pl.when(i == 0)
    def _():
        gsum[...] = jnp.zeros_like(gsum)
        cnt[...] = jnp.zeros_like(cnt)

    agg = jnp.concatenate([aA_ref[...], aB_ref[...]], axis=1)
    msp = jnp.concatenate([mA_ref[...], mB_ref[...]], axis=1)
    h = jnp.maximum(dinv_ref[...] * (agg + msp) + b_ref[...], 0.0)
    gid = batch_ref[...]
    oh = (gid == lax.broadcasted_iota(jnp.int32, (BM, GG), 1)).astype(jnp.float32)
    dn = (((0,), (0,)), ((), ()))
    gsum[...] += lax.dot_general(oh, h, dn, preferred_element_type=jnp.float32)
    cnt[...] += lax.dot_general(oh, jnp.ones((BM, 1), jnp.float32), dn,
                                preferred_element_type=jnp.float32)

    @pl.when(i == NBLK - 1)
    def _():
        gmean = gsum[...] / jnp.maximum(cnt[...], 1.0)
        z = jnp.maximum(jnp.dot(gmean, wc1_ref[...],
                                preferred_element_type=jnp.float32) + bc1_ref[...], 0.0)
        o = jnp.dot(z, wc2_ref[...], preferred_element_type=jnp.float32) + bc2_ref[...]
        out_ref[...] = jax.nn.sigmoid(o)


_stage_c = pl.pallas_call(
    _stage_c_body,
    grid=(NBLK,),
    in_specs=[_hhalf, _hhalf, _hhalf, _hhalf, _col, _full((1, HH)), _col,
              _full((HH, HALF)), _full((1, HALF)), _full((HALF, 1)), _full((1, 1))],
    out_specs=pl.BlockSpec((GG, 1), lambda i: (0, 0)),
    out_shape=jax.ShapeDtypeStruct((GG, 1), jnp.float32),
    scratch_shapes=[pltpu.VMEM((GG, HH), jnp.float32), pltpu.VMEM((GG, 1), jnp.float32)],
)


def kernel(x, edge_index, batch, W_embed, b_embed, W_gcn, b_gcn, Wc1, bc1, Wc2, bc2):
    _sc_deg, _sc_agg = _build_sc()
    ei_flat = edge_index.reshape(2 * EE)
    deg0, deg1 = _sc_deg(ei_flat)
    msA, msB, dinv = _stage_a(
        x, deg0.reshape(NN, 1), deg1.reshape(NN, 1),
        W_embed.reshape(1, HH), b_embed.reshape(1, HH), W_gcn[0])
    for i in (1, 2):
        aggA, aggB = _sc_agg(ei_flat, msA, msB)
        msA, msB = _stage_b(aggA, aggB, msA, msB, dinv,
                            b_gcn[i - 1].reshape(1, HH), W_gcn[i])
    aggA, aggB = _sc_agg(ei_flat, msA, msB)
    out = _stage_c(aggA, aggB, msA, msB, dinv, b_gcn[2].reshape(1, HH),
                   batch.reshape(NN, 1), Wc1, bc1.reshape(1, HALF),
                   Wc2, bc2.reshape(1, 1))
    return out.reshape(GG)


# speedup vs baseline: 30.0436x; 1.0139x over previous
import functools

import jax
import jax.numpy as jnp
from jax import lax
from jax.experimental import pallas as pl
from jax.experimental.pallas import tpu as pltpu
from jax.experimental.pallas import tpu_sc as plsc

NN = 50000
EE = 800000
HH = 64
HALF = 32
GG = 64
NT = 16
NC = 2

BM = 5000
NBLK = NN // BM

DEG_PER_W = EE // (NC * NT)
DEG_FULL = DEG_PER_W // 128
DEG_TAIL = DEG_PER_W - DEG_FULL * 128

AGG_PER_T = EE // NT
AGG_FULL = AGG_PER_T // 128
AGG_TAIL = AGG_PER_T - AGG_FULL * 128

ROWS_PER_T = NN // NT

def _zero_vmem_2d(buf, rows):
    def body(i, _):
        buf[i, pl.ds(0, 16)] = jnp.zeros((16,), jnp.float32)
        buf[i, pl.ds(16, 16)] = jnp.zeros((16,), jnp.float32)
        return 0
    lax.fori_loop(0, rows, body, 0)


def _zero_vmem_1d(buf, n16):
    def body(i, _):
        buf[pl.ds(i * 16, 16)] = jnp.zeros((16,), jnp.float32)
        return 0
    lax.fori_loop(0, n16, body, 0)


def _sc_deg_body(ei, out0, out1, idx, idx_t, ones, zbuf, acc):
    c = lax.axis_index("c")
    s = lax.axis_index("s")
    _zero_vmem_1d(zbuf, 200)
    def ob(i, _):
        ones[pl.ds(i * 16, 16)] = jnp.ones((16,), jnp.float32)
        return 0
    lax.fori_loop(0, 8, ob, 0)
    @pl.when(s < NT - 1)
    def _():
        pltpu.sync_copy(zbuf, acc.at[pl.ds(s * 3200, 3200)])
    @pl.when(s == NT - 1)
    def _():
        pltpu.sync_copy(zbuf.at[pl.ds(0, 2000)], acc.at[pl.ds(48000, 2000)])
    plsc.subcore_barrier()

    wbase = (c * NT + s) * DEG_PER_W

    def dslab(j, _):
        pltpu.sync_copy(ei.at[pl.ds(EE + wbase + j * 1920, 1920)], idx)
        def chunk(k, _):
            pltpu.sync_copy(ones, acc.at[idx.at[pl.ds(k * 128, 128)]], add=True)
            return 0
        lax.fori_loop(0, 15, chunk, 0)
        return 0
    lax.fori_loop(0, 13, dslab, 0)
    pltpu.sync_copy(ei.at[pl.ds(EE + wbase + 24960, DEG_TAIL)], idx_t)
    pltpu.sync_copy(ones.at[pl.ds(0, DEG_TAIL)], acc.at[idx_t], add=True)
    plsc.subcore_barrier()

    def wb(out):
        @pl.when(s < NT - 1)
        def _():
            pltpu.sync_copy(acc.at[pl.ds(s * 3200, 3200)], zbuf)
            pltpu.sync_copy(zbuf, out.at[pl.ds(s * 3200, 3200)])
        @pl.when(s == NT - 1)
        def _():
            pltpu.sync_copy(acc.at[pl.ds(48000, 2000)], zbuf.at[pl.ds(0, 2000)])
            pltpu.sync_copy(zbuf.at[pl.ds(0, 2000)], out.at[pl.ds(48000, 2000)])
    @pl.when(c == 0)
    def _():
        wb(out0)
    @pl.when(c == 1)
    def _():
        wb(out1)


NBUF = 4
NCHUNK = EE // 128
CPT = NCHUNK // NT
XTRA = NCHUNK - CPT * NT
SLAB = 15


def _sc_agg_body(ei, msA, msB, outA, outB,
                 sidx, didx, dbuf, rows, zbuf, acc, sem_g, sem_s, sem_i, sem_e):
    c = lax.axis_index("c")
    s = lax.axis_index("s")
    _zero_vmem_2d(zbuf, 125)
    r0 = s * ROWS_PER_T
    def zb(j, _):
        pltpu.sync_copy(zbuf, acc.at[pl.ds(r0 + j * 125, 125), :])
        return 0
    lax.fori_loop(0, ROWS_PER_T // 125, zb, 0)
    plsc.subcore_barrier()

    cs = s * CPT + jnp.minimum(s, XTRA)
    NSLAB = CPT // SLAB

    def issue_slab(j):
        p = lax.rem(j, 2)
        cb = cs + j * SLAB
        pltpu.async_copy(ei.at[pl.ds(cb * 128, SLAB * 128)], sidx.at[p], sem_i.at[p])
        pltpu.async_copy(ei.at[pl.ds(EE + cb * 128, SLAB * 128)], didx.at[p], sem_i.at[p])

    def wait_slab(j):
        p = lax.rem(j, 2)
        cb = cs + j * SLAB
        pltpu.make_async_copy(ei.at[pl.ds(cb * 128, SLAB * 128)], sidx.at[p],
                              sem_i.at[p]).wait()
        pltpu.make_async_copy(ei.at[pl.ds(EE + cb * 128, SLAB * 128)], didx.at[p],
                              sem_i.at[p]).wait()

    def issue_gather(p, i):
        r = lax.rem(i, NBUF)
        @pl.when(c == 0)
        def _():
            pltpu.async_copy(msA.at[sidx.at[p, pl.ds(i * 128, 128)]], rows.at[r],
                             sem_g.at[r])
        @pl.when(c == 1)
        def _():
            pltpu.async_copy(msB.at[sidx.at[p, pl.ds(i * 128, 128)]], rows.at[r],
                             sem_g.at[r])

    def wait_gather(p, i):
        r = lax.rem(i, NBUF)
        pltpu.make_async_copy(msA.at[sidx.at[p, pl.ds(i * 128, 128)]], rows.at[r],
                              sem_g.at[r]).wait()

    def load_dbuf(p, i):
        r = lax.rem(i, NBUF)
        for k in range(8):
            dbuf[r, pl.ds(k * 16, 16)] = didx[p, pl.ds(i * 128 + k * 16, 16)]

    def issue_scatter(i):
        r = lax.rem(i, NBUF)
        pltpu.async_copy(rows.at[r], acc.at[dbuf.at[r]], sem_s.at[r], add=True)

    def wait_scatter(i):
        r = lax.rem(i, NBUF)
        pltpu.make_async_copy(rows.at[r], acc.at[dbuf.at[r]], sem_s.at[r]).wait()

    issue_slab(0)

    def slab(j, _):
        p = lax.rem(j, 2)
        wait_slab(j)
        @pl.when(j + 1 < NSLAB)
        def _():
            issue_slab(j + 1)
        for q in range(NBUF):
            issue_gather(p, q)
        def body(i, _):
            wait_gather(p, i)
            load_dbuf(p, i)
            issue_scatter(i)
            @pl.when(i + NBUF < SLAB)
            def _():
                wait_scatter(i)
                issue_gather(p, i + NBUF)
            return 0
        lax.fori_loop(0, SLAB, body, 0)
        for q in range(SLAB - NBUF, SLAB):
            wait_scatter(q)
        return 0
    lax.fori_loop(0, NSLAB, slab, 0)

    @pl.when(s < XTRA)
    def _():
        ce = cs + CPT
        pltpu.sync_copy(ei.at[pl.ds(ce * 128, 128)], sidx.at[0, pl.ds(0, 128)])
        pltpu.sync_copy(ei.at[pl.ds(EE + ce * 128, 128)], didx.at[0, pl.ds(0, 128)])
        @pl.when(c == 0)
        def _():
            pltpu.async_copy(msA.at[sidx.at[0, pl.ds(0, 128)]], rows.at[0], sem_e).wait()
        @pl.when(c == 1)
        def _():
            pltpu.async_copy(msB.at[sidx.at[0, pl.ds(0, 128)]], rows.at[0], sem_e).wait()
        load_dbuf(0, 0)
        pltpu.sync_copy(rows.at[0], acc.at[dbuf.at[0]], add=True)
    plsc.subcore_barrier()

    def wb(out):
        @pl.when(s < NT - 1)
        def _():
            pltpu.sync_copy(acc.at[pl.ds(s * 3200, 3200), :], out.at[pl.ds(s * 3200, 3200), :])
        @pl.when(s == NT - 1)
        def _():
            pltpu.sync_copy(acc.at[pl.ds(48000, 2000), :], out.at[pl.ds(48000, 2000), :])
    @pl.when(c == 0)
    def _():
        wb(outA)
    @pl.when(c == 1)
    def _():
        wb(outB)


@functools.lru_cache(maxsize=None)
def _build_sc():
    mesh = plsc.VectorSubcoreMesh(core_axis_name="c", subcore_axis_name="s")
    params = pltpu.CompilerParams(use_tc_tiling_on_sc=False)
    sc_deg = pl.kernel(
        _sc_deg_body,
        out_type=[jax.ShapeDtypeStruct((NN,), jnp.float32) for _ in range(NC)],
        mesh=mesh,
        compiler_params=params,
        scratch_types=[
            pltpu.VMEM((1920,), jnp.int32),
            pltpu.VMEM((DEG_TAIL,), jnp.int32),
            pltpu.VMEM((128,), jnp.float32),
            pltpu.VMEM((3200,), jnp.float32),
            pltpu.VMEM_SHARED((NN,), jnp.float32),
        ],
    )
    sc_agg = pl.kernel(
        _sc_agg_body,
        out_type=[jax.ShapeDtypeStruct((NN, HALF), jnp.float32) for _ in range(NC)],
        mesh=mesh,
        compiler_params=params,
        scratch_types=[
            pltpu.VMEM((2, SLAB * 128), jnp.int32),
            pltpu.VMEM((2, SLAB * 128), jnp.int32),
            pltpu.VMEM((NBUF, 128), jnp.int32),
            pltpu.VMEM((NBUF, 128, HALF), jnp.float32),
            pltpu.VMEM((125, HALF), jnp.float32),
            pltpu.VMEM_SHARED((NN, HALF), jnp.float32),
            pltpu.SemaphoreType.DMA((NBUF,)),
            pltpu.SemaphoreType.DMA((NBUF,)),
            pltpu.SemaphoreType.DMA((2,)),
            pltpu.SemaphoreType.DMA,
        ],
    )
    return sc_deg, sc_agg


def _stage_a_body(x_ref, d0_ref, d1_ref, we_ref, be_ref, w0_ref,
                  msA_ref, msB_ref, dinv_ref):
    deg = d0_ref[...] + d1_ref[...] + 1.0
    dinv = lax.rsqrt(deg)
    h0 = x_ref[...] * we_ref[...] + be_ref[...]
    m = jnp.dot(h0, w0_ref[...], preferred_element_type=jnp.float32)
    ms = dinv * m
    msA_ref[...] = ms[:, :HALF]
    msB_ref[...] = ms[:, HALF:]
    dinv_ref[...] = dinv


_col = pl.BlockSpec((BM, 1), lambda i: (i, 0))
_hhalf = pl.BlockSpec((BM, HALF), lambda i: (i, 0))
_full = lambda shape: pl.BlockSpec(shape, lambda i: tuple(0 for _ in shape))

_stage_a = pl.pallas_call(
    _stage_a_body,
    grid=(NBLK,),
    in_specs=[_col, _col, _col, _full((1, HH)), _full((1, HH)), _full((HH, HH))],
    out_specs=[_hhalf, _hhalf, _col],
    out_shape=[
        jax.ShapeDtypeStruct((NN, HALF), jnp.float32),
        jax.ShapeDtypeStruct((NN, HALF), jnp.float32),
        jax.ShapeDtypeStruct((NN, 1), jnp.float32),
    ],
)


def _stage_b_body(aA_ref, aB_ref, mA_ref, mB_ref, dinv_ref, b_ref, w_ref,
                  oA_ref, oB_ref):
    agg = jnp.concatenate([aA_ref[...], aB_ref[...]], axis=1)
    msp = jnp.concatenate([mA_ref[...], mB_ref[...]], axis=1)
    dinv = dinv_ref[...]
    h = jnp.maximum(dinv * (agg + msp) + b_ref[...], 0.0)
    ms = dinv * jnp.dot(h, w_ref[...], preferred_element_type=jnp.float32)
    oA_ref[...] = ms[:, :HALF]
    oB_ref[...] = ms[:, HALF:]


_stage_b = pl.pallas_call(
    _stage_b_body,
    grid=(NBLK,),
    in_specs=[_hhalf, _hhalf, _hhalf, _hhalf, _col, _full((1, HH)), _full((HH, HH))],
    out_specs=[_hhalf, _hhalf],
    out_shape=[
        jax.ShapeDtypeStruct((NN, HALF), jnp.float32),
        jax.ShapeDtypeStruct((NN, HALF), jnp.float32),
    ],
)


def _stage_c_body(aA_ref, aB_ref, mA_ref, mB_ref, dinv_ref, b_ref, batch_ref,
                  wc1_ref, bc1_ref, wc2_ref, bc2_ref, out_ref, gsum, cnt):
    i = pl.program_id(0)

    @pl.when(i == 0)
    def _():
        gsum[...] = jnp.zeros_like(gsum)
        cnt[...] = jnp.zeros_like(cnt)

    agg = jnp.concatenate([aA_ref[...], aB_ref[...]], axis=1)
    msp = jnp.concatenate([mA_ref[...], mB_ref[...]], axis=1)
    h = jnp.maximum(dinv_ref[...] * (agg + msp) + b_ref[...], 0.0)
    gid = batch_ref[...]
    oh = (gid == lax.broadcasted_iota(jnp.int32, (BM, GG), 1)).astype(jnp.float32)
    dn = (((0,), (0,)), ((), ()))
    gsum[...] += lax.dot_general(oh, h, dn, preferred_element_type=jnp.float32)
    cnt[...] += lax.dot_general(oh, jnp.ones((BM, 1), jnp.float32), dn,
                                preferred_element_type=jnp.float32)

    @pl.when(i == NBLK - 1)
    def _():
        gmean = gsum[...] / jnp.maximum(cnt[...], 1.0)
        z = jnp.maximum(jnp.dot(gmean, wc1_ref[...],
                                preferred_element_type=jnp.float32) + bc1_ref[...], 0.0)
        o = jnp.dot(z, wc2_ref[...], preferred_element_type=jnp.float32) + bc2_ref[...]
        out_ref[...] = jax.nn.sigmoid(o)


_stage_c = pl.pallas_call(
    _stage_c_body,
    grid=(NBLK,),
    in_specs=[_hhalf, _hhalf, _hhalf, _hhalf, _col, _full((1, HH)), _col,
              _full((HH, HALF)), _full((1, HALF)), _full((HALF, 1)), _full((1, 1))],
    out_specs=pl.BlockSpec((GG, 1), lambda i: (0, 0)),
    out_shape=jax.ShapeDtypeStruct((GG, 1), jnp.float32),
    scratch_shapes=[pltpu.VMEM((GG, HH), jnp.float32), pltpu.VMEM((GG, 1), jnp.float32)],
)


def kernel(x, edge_index, batch, W_embed, b_embed, W_gcn, b_gcn, Wc1, bc1, Wc2, bc2):
    _sc_deg, _sc_agg = _build_sc()
    ei_flat = edge_index.reshape(2 * EE)
    deg0, deg1 = _sc_deg(ei_flat)
    msA, msB, dinv = _stage_a(
        x, deg0.reshape(NN, 1), deg1.reshape(NN, 1),
        W_embed.reshape(1, HH), b_embed.reshape(1, HH), W_gcn[0])
    for i in (1, 2):
        aggA, aggB = _sc_agg(ei_flat, msA, msB)
        msA, msB = _stage_b(aggA, aggB, msA, msB, dinv,
                            b_gcn[i - 1].reshape(1, HH), W_gcn[i])
    aggA, aggB = _sc_agg(ei_flat, msA, msB)
    out = _stage_c(aggA, aggB, msA, msB, dinv, b_gcn[2].reshape(1, HH),
                   batch.reshape(NN, 1), Wc1, bc1.reshape(1, HALF),
                   Wc2, bc2.reshape(1, 1))
    return out.reshape(GG)
